# Initial kernel scaffold; baseline (speedup 1.0000x reference)
import jax
import jax.numpy as jnp
from jax.experimental import pallas as pl


def kernel(x, edge_index, edge_attr, W1, root1, bias1, W2, root2, bias2):
    raise NotImplementedError("write your pallas kernel here")


# speedup vs baseline: 24.5847x; 24.5847x over previous
import functools

import jax
import jax.numpy as jnp
from jax import lax
from jax.experimental import pallas as pl
from jax.experimental.pallas import tpu as pltpu
from jax.experimental.pallas import tpu_sc as plsc

N = 10000
E = 320000
R = 4
D = 128
NRPAD = 40960
NC, NS = 2, 16
NW = NC * NS
EPT = E // NW
K = 80
NCHUNK = EPT // K
NPAD = 10240
ROWS_PT = NPAD // NS

_mesh = plsc.VectorSubcoreMesh(core_axis_name="c", subcore_axis_name="s")
_sc_params = pltpu.CompilerParams(needs_layout_passes=False)


def _mm_body(x_ref, w_ref, o_ref):
    o_ref[0] = jnp.dot(x_ref[...], w_ref[0], preferred_element_type=jnp.float32)


def _mm(x, wc):
    bm = 400
    return pl.pallas_call(
        _mm_body,
        grid=(R + 1, N // bm),
        in_specs=[
            pl.BlockSpec((bm, D), lambda r, j: (j, 0)),
            pl.BlockSpec((1, D, D), lambda r, j: (r, 0, 0)),
        ],
        out_specs=pl.BlockSpec((1, bm, D), lambda r, j: (r, j, 0)),
        out_shape=jax.ShapeDtypeStruct((R + 1, N, D), jnp.float32),
    )(x, wc)


def _c16(v):
    return jnp.full((16,), v, jnp.int32)


def _counts_body(pk_hbm, out_hbm, pk_v, cnt_v):
    cid = lax.axis_index("c")
    sid = lax.axis_index("s")
    wid = cid * NS + sid
    base = wid * EPT

    def zero_body(i, _):
        cnt_v[pl.ds(i * 16, 16)] = jnp.zeros((16,), jnp.float32)
        return 0

    lax.fori_loop(0, NRPAD // 16, zero_body, 0)

    lane_io = lax.broadcasted_iota(jnp.int32, (16,), 0)
    ones = jnp.ones((16,), jnp.float32)

    def chunk_body(j, _):
        off = base + j * K
        pltpu.sync_copy(pk_hbm.at[pl.ds(off, K)], pk_v)
        for k in range(K // 16):
            sl = pl.ds(16 * k, 16)
            cidx = lax.shift_right_logical(pk_v[sl], _c16(16))
            for lane in range(16):
                plsc.addupdate_scatter(cnt_v, [cidx], ones,
                                       mask=lane_io == lane)
        return 0

    lax.fori_loop(0, NCHUNK, chunk_body, 0)
    pltpu.sync_copy(cnt_v, out_hbm.at[wid])


@functools.partial(
    pl.kernel,
    out_type=jax.ShapeDtypeStruct((NW, NRPAD), jnp.float32),
    mesh=_mesh,
    scratch_types=[
        pltpu.VMEM((K,), jnp.int32),
        pltpu.VMEM((NRPAD,), jnp.float32),
    ],
    compiler_params=_sc_params,
)
def _sc_counts(pk_hbm, out_hbm, pk_v, cnt_v):
    _counts_body(pk_hbm, out_hbm, pk_v, cnt_v)


def _inv_body(c_ref, o_ref):
    s = jnp.sum(c_ref[...], axis=0)
    o_ref[...] = 1.0 / jnp.maximum(s, 1.0)


def _inv(counts):
    c3 = counts.reshape(NW, NRPAD // 128, 128)
    out = pl.pallas_call(
        _inv_body,
        grid=(NRPAD // (8 * 128),),
        in_specs=[pl.BlockSpec((NW, 8, 128), lambda i: (0, i, 0))],
        out_specs=pl.BlockSpec((8, 128), lambda i: (i, 0)),
        out_shape=jax.ShapeDtypeStruct((NRPAD // 128, 128), jnp.float32),
    )(c3)
    return out.reshape(NRPAD)


def _norm_body(pk_hbm, inv_hbm, out_hbm, pk_v, norm_v, inv_v):
    cid = lax.axis_index("c")
    sid = lax.axis_index("s")
    wid = cid * NS + sid
    base = wid * EPT

    pltpu.sync_copy(inv_hbm, inv_v)

    def chunk_body(j, _):
        off = base + j * K
        pltpu.sync_copy(pk_hbm.at[pl.ds(off, K)], pk_v)
        for k in range(K // 16):
            sl = pl.ds(16 * k, 16)
            cidx = lax.shift_right_logical(pk_v[sl], _c16(16))
            norm_v[sl] = plsc.load_gather(inv_v, [cidx])
        pltpu.sync_copy(norm_v, out_hbm.at[pl.ds(off, K)])
        return 0

    lax.fori_loop(0, NCHUNK, chunk_body, 0)


@functools.partial(
    pl.kernel,
    out_type=jax.ShapeDtypeStruct((E,), jnp.float32),
    mesh=_mesh,
    scratch_types=[
        pltpu.VMEM((K,), jnp.int32),
        pltpu.VMEM((K,), jnp.float32),
        pltpu.VMEM((NRPAD,), jnp.float32),
    ],
    compiler_params=_sc_params,
)
def _sc_norm(pk_hbm, inv_hbm, out_hbm, pk_v, norm_v, inv_v):
    _norm_body(pk_hbm, inv_hbm, out_hbm, pk_v, norm_v, inv_v)


def _msg_body(pk_hbm, nrm_hbm, tab_hbm, out_hbm,
              pk_v, dst_v, gidx_v, norm_v, rows_v,
              agg_sp, sem):
    cid = lax.axis_index("c")
    sid = lax.axis_index("s")
    wid = cid * NS + sid
    base = wid * EPT

    def zb(i, _):
        for k in range(8):
            rows_v[i, pl.ds(16 * k, 16)] = jnp.zeros((16,), jnp.float32)
        return 0

    lax.fori_loop(0, K, zb, 0)
    for j in range(ROWS_PT // K):
        pltpu.sync_copy(rows_v, agg_sp.at[pl.ds(sid * ROWS_PT + j * K, K)])
    plsc.subcore_barrier()

    def chunk_body(j, _):
        off = base + j * K
        pltpu.sync_copy(pk_hbm.at[pl.ds(off, K)], pk_v)
        pltpu.sync_copy(nrm_hbm.at[pl.ds(off, K)], norm_v)
        for k in range(K // 16):
            sl = pl.ds(16 * k, 16)
            w = pk_v[sl]
            t = lax.shift_right_logical(w, _c16(16)) & _c16(3)
            gidx_v[sl] = t * N + (w & _c16(0x3FFF))
            dst_v[sl] = lax.shift_right_logical(w, _c16(18))
        pltpu.async_copy(tab_hbm.at[gidx_v], rows_v, sem).wait()

        def scale_body(e, _):
            bc = plsc.load_gather(norm_v, [jnp.broadcast_to(e, (16,))])
            for k in range(8):
                sl = pl.ds(16 * k, 16)
                rows_v[e, sl] = rows_v[e, sl] * bc
            return 0

        lax.fori_loop(0, K, scale_body, 0)
        pltpu.sync_copy(rows_v, agg_sp.at[dst_v], add=True)
        return 0

    lax.fori_loop(0, NCHUNK, chunk_body, 0)
    plsc.subcore_barrier()
    pltpu.sync_copy(agg_sp.at[pl.ds(sid * ROWS_PT, ROWS_PT)],
                    out_hbm.at[cid, pl.ds(sid * ROWS_PT, ROWS_PT)])


@functools.partial(
    pl.kernel,
    out_type=jax.ShapeDtypeStruct((NC, NPAD, D), jnp.float32),
    mesh=_mesh,
    scratch_types=[
        pltpu.VMEM((K,), jnp.int32),
        pltpu.VMEM((K,), jnp.int32),
        pltpu.VMEM((K,), jnp.int32),
        pltpu.VMEM((K,), jnp.float32),
        pltpu.VMEM((K, D), jnp.float32),
        pltpu.VMEM_SHARED((NPAD, D), jnp.float32),
        pltpu.SemaphoreType.DMA,
    ],
    compiler_params=_sc_params,
)
def _sc_msg(pk_hbm, nrm_hbm, tab_hbm, out_hbm,
            pk_v, dst_v, gidx_v, norm_v, rows_v,
            agg_sp, sem):
    _msg_body(pk_hbm, nrm_hbm, tab_hbm, out_hbm,
              pk_v, dst_v, gidx_v, norm_v, rows_v,
              agg_sp, sem)


def _fin_body(a_ref, xr_ref, b_ref, o_ref):
    o_ref[...] = jnp.tanh(a_ref[0] + a_ref[1] + xr_ref[...] + b_ref[...])


def _finish(aggp, xroot, bias):
    bm = 400
    return pl.pallas_call(
        _fin_body,
        grid=(N // bm,),
        in_specs=[
            pl.BlockSpec((NC, bm, D), lambda i: (0, i, 0)),
            pl.BlockSpec((bm, D), lambda i: (i, 0)),
            pl.BlockSpec((1, D), lambda i: (0, 0)),
        ],
        out_specs=pl.BlockSpec((bm, D), lambda i: (i, 0)),
        out_shape=jax.ShapeDtypeStruct((N, D), jnp.float32),
    )(aggp, xroot, bias.reshape(1, D))


def _layer(h, pk, nrm, W, root, bias):
    wc = jnp.concatenate([W, root[None]], axis=0)
    xw = _mm(h, wc)
    table = xw.reshape((R + 1) * N, D)
    aggp = _sc_msg(pk, nrm, table)
    return _finish(aggp, xw[R], bias)


def kernel(x, edge_index, edge_attr, W1, root1, bias1, W2, root2, bias2):
    src = edge_index[0]
    dst = edge_index[1]
    pk = (dst << 18) | (edge_attr << 16) | src
    counts = _sc_counts(pk)
    inv = _inv(counts)
    nrm = _sc_norm(pk, inv)
    h = _layer(x, pk, nrm, W1, root1, bias1)
    return _layer(h, pk, nrm, W2, root2, bias2)


# speedup vs baseline: 31.6956x; 1.2892x over previous
import functools

import jax
import jax.numpy as jnp
from jax import lax
from jax.experimental import pallas as pl
from jax.experimental.pallas import tpu as pltpu
from jax.experimental.pallas import tpu_sc as plsc

N = 10000
E = 320000
R = 4
D = 128
NRPAD = 40960
NC, NS = 2, 16
NW = NC * NS
EPT = E // NW
K = 80
NCHUNK = EPT // K
NPAD = 10240
ROWS_PT = NPAD // NS

_mesh = plsc.VectorSubcoreMesh(core_axis_name="c", subcore_axis_name="s")
_sc_params = pltpu.CompilerParams(needs_layout_passes=False)


def _mm_body(x_ref, w_ref, o_ref):
    o_ref[0] = jnp.dot(x_ref[...], w_ref[0], preferred_element_type=jnp.float32)


def _mm(x, wc):
    bm = 400
    return pl.pallas_call(
        _mm_body,
        grid=(R + 1, N // bm),
        in_specs=[
            pl.BlockSpec((bm, D), lambda r, j: (j, 0)),
            pl.BlockSpec((1, D, D), lambda r, j: (r, 0, 0)),
        ],
        out_specs=pl.BlockSpec((1, bm, D), lambda r, j: (r, j, 0)),
        out_shape=jax.ShapeDtypeStruct((R + 1, N, D), jnp.float32),
    )(x, wc)


def _c16(v):
    return jnp.full((16,), v, jnp.int32)


def _counts_body(pk_hbm, out_hbm, pk_v, cnt_v):
    cid = lax.axis_index("c")
    sid = lax.axis_index("s")
    wid = cid * NS + sid
    base = wid * EPT

    def zero_body(i, _):
        cnt_v[pl.ds(i * 16, 16)] = jnp.zeros((16,), jnp.float32)
        return 0

    lax.fori_loop(0, NRPAD // 16, zero_body, 0)

    lane_io = lax.broadcasted_iota(jnp.int32, (16,), 0)
    ones = jnp.ones((16,), jnp.float32)

    def chunk_body(j, _):
        off = base + j * K
        pltpu.sync_copy(pk_hbm.at[pl.ds(off, K)], pk_v)
        for k in range(K // 16):
            sl = pl.ds(16 * k, 16)
            cidx = lax.shift_right_logical(pk_v[sl], _c16(16))
            for lane in range(16):
                plsc.addupdate_scatter(cnt_v, [cidx], ones,
                                       mask=lane_io == lane)
        return 0

    lax.fori_loop(0, NCHUNK, chunk_body, 0)
    pltpu.sync_copy(cnt_v, out_hbm.at[wid])


@functools.partial(
    pl.kernel,
    out_type=jax.ShapeDtypeStruct((NW, NRPAD), jnp.float32),
    mesh=_mesh,
    scratch_types=[
        pltpu.VMEM((K,), jnp.int32),
        pltpu.VMEM((NRPAD,), jnp.float32),
    ],
    compiler_params=_sc_params,
)
def _sc_counts(pk_hbm, out_hbm, pk_v, cnt_v):
    _counts_body(pk_hbm, out_hbm, pk_v, cnt_v)


def _inv_body(c_ref, o_ref):
    s = jnp.sum(c_ref[...], axis=0)
    o_ref[...] = 1.0 / jnp.maximum(s, 1.0)


def _inv(counts):
    c3 = counts.reshape(NW, NRPAD // 128, 128)
    out = pl.pallas_call(
        _inv_body,
        grid=(NRPAD // (8 * 128),),
        in_specs=[pl.BlockSpec((NW, 8, 128), lambda i: (0, i, 0))],
        out_specs=pl.BlockSpec((8, 128), lambda i: (i, 0)),
        out_shape=jax.ShapeDtypeStruct((NRPAD // 128, 128), jnp.float32),
    )(c3)
    return out.reshape(NRPAD)


def _norm_body(pk_hbm, inv_hbm, out_hbm, pk_v, norm_v, inv_v):
    cid = lax.axis_index("c")
    sid = lax.axis_index("s")
    wid = cid * NS + sid
    base = wid * EPT

    pltpu.sync_copy(inv_hbm, inv_v)

    def chunk_body(j, _):
        off = base + j * K
        pltpu.sync_copy(pk_hbm.at[pl.ds(off, K)], pk_v)
        for k in range(K // 16):
            sl = pl.ds(16 * k, 16)
            cidx = lax.shift_right_logical(pk_v[sl], _c16(16))
            norm_v[sl] = plsc.load_gather(inv_v, [cidx])
        pltpu.sync_copy(norm_v, out_hbm.at[pl.ds(off, K)])
        return 0

    lax.fori_loop(0, NCHUNK, chunk_body, 0)


@functools.partial(
    pl.kernel,
    out_type=jax.ShapeDtypeStruct((E,), jnp.float32),
    mesh=_mesh,
    scratch_types=[
        pltpu.VMEM((K,), jnp.int32),
        pltpu.VMEM((K,), jnp.float32),
        pltpu.VMEM((NRPAD,), jnp.float32),
    ],
    compiler_params=_sc_params,
)
def _sc_norm(pk_hbm, inv_hbm, out_hbm, pk_v, norm_v, inv_v):
    _norm_body(pk_hbm, inv_hbm, out_hbm, pk_v, norm_v, inv_v)


def _msg_body(pk_hbm, nrm_hbm, tab_hbm, out_hbm,
              pk_v, dst_v, gidx_v, norm_v, rows_v, sems,
              agg_sp):
    cid = lax.axis_index("c")
    sid = lax.axis_index("s")
    wid = cid * NS + sid
    base = wid * EPT

    def zb(i, _):
        for k in range(8):
            rows_v[0][i, pl.ds(16 * k, 16)] = jnp.zeros((16,), jnp.float32)
        return 0

    lax.fori_loop(0, K, zb, 0)
    for j in range(ROWS_PT // K):
        pltpu.sync_copy(rows_v[0], agg_sp.at[pl.ds(sid * ROWS_PT + j * K, K)])
    plsc.subcore_barrier()

    def load_idx(j, b):
        off = base + j * K
        pltpu.sync_copy(pk_hbm.at[pl.ds(off, K)], pk_v[b])
        pltpu.sync_copy(nrm_hbm.at[pl.ds(off, K)], norm_v[b])
        for k in range(K // 16):
            sl = pl.ds(16 * k, 16)
            w = pk_v[b][sl]
            t = lax.shift_right_logical(w, _c16(16)) & _c16(3)
            gidx_v[b][sl] = t * N + (w & _c16(0x3FFF))
            dst_v[b][sl] = lax.shift_right_logical(w, _c16(18))

    def fire(b):
        pltpu.async_copy(tab_hbm.at[gidx_v[b]], rows_v[b], sems[b])

    def finish(b):
        pltpu.make_async_copy(tab_hbm.at[gidx_v[b]], rows_v[b],
                              sems[b]).wait()

        def scale_body(e, _):
            bc = plsc.load_gather(norm_v[b], [jnp.broadcast_to(e, (16,))])
            for k in range(8):
                sl = pl.ds(16 * k, 16)
                rows_v[b][e, sl] = rows_v[b][e, sl] * bc
            return 0

        lax.fori_loop(0, K, scale_body, 0)
        pltpu.sync_copy(rows_v[b], agg_sp.at[dst_v[b]], add=True)

    load_idx(0, 0)
    fire(0)

    def pair_body(i, _):
        j = 2 * i
        load_idx(j + 1, 1)
        fire(1)
        finish(0)
        load_idx(j + 2, 0)
        fire(0)
        finish(1)
        return 0

    lax.fori_loop(0, (NCHUNK - 1) // 2, pair_body, 0)
    finish(0)

    plsc.subcore_barrier()
    pltpu.sync_copy(agg_sp.at[pl.ds(sid * ROWS_PT, ROWS_PT)],
                    out_hbm.at[cid, pl.ds(sid * ROWS_PT, ROWS_PT)])


@functools.partial(
    pl.kernel,
    out_type=jax.ShapeDtypeStruct((NC, NPAD, D), jnp.float32),
    mesh=_mesh,
    scratch_types=[
        [pltpu.VMEM((K,), jnp.int32)] * 2,
        [pltpu.VMEM((K,), jnp.int32)] * 2,
        [pltpu.VMEM((K,), jnp.int32)] * 2,
        [pltpu.VMEM((K,), jnp.float32)] * 2,
        [pltpu.VMEM((K, D), jnp.float32)] * 2,
        [pltpu.SemaphoreType.DMA] * 2,
        pltpu.VMEM_SHARED((NPAD, D), jnp.float32),
    ],
    compiler_params=_sc_params,
)
def _sc_msg(pk_hbm, nrm_hbm, tab_hbm, out_hbm,
            pk_v, dst_v, gidx_v, norm_v, rows_v, sems,
            agg_sp):
    _msg_body(pk_hbm, nrm_hbm, tab_hbm, out_hbm,
              pk_v, dst_v, gidx_v, norm_v, rows_v, sems,
              agg_sp)


def _fin_body(a_ref, xr_ref, b_ref, o_ref):
    o_ref[...] = jnp.tanh(a_ref[0] + a_ref[1] + xr_ref[...] + b_ref[...])


def _finish(aggp, xroot, bias):
    bm = 400
    return pl.pallas_call(
        _fin_body,
        grid=(N // bm,),
        in_specs=[
            pl.BlockSpec((NC, bm, D), lambda i: (0, i, 0)),
            pl.BlockSpec((bm, D), lambda i: (i, 0)),
            pl.BlockSpec((1, D), lambda i: (0, 0)),
        ],
        out_specs=pl.BlockSpec((bm, D), lambda i: (i, 0)),
        out_shape=jax.ShapeDtypeStruct((N, D), jnp.float32),
    )(aggp, xroot, bias.reshape(1, D))


def _layer(h, pk, nrm, W, root, bias):
    wc = jnp.concatenate([W, root[None]], axis=0)
    xw = _mm(h, wc)
    table = xw.reshape((R + 1) * N, D)
    aggp = _sc_msg(pk, nrm, table)
    return _finish(aggp, xw[R], bias)


def kernel(x, edge_index, edge_attr, W1, root1, bias1, W2, root2, bias2):
    src = edge_index[0]
    dst = edge_index[1]
    pk = (dst << 18) | (edge_attr << 16) | src
    counts = _sc_counts(pk)
    inv = _inv(counts)
    nrm = _sc_norm(pk, inv)
    h = _layer(x, pk, nrm, W1, root1, bias1)
    return _layer(h, pk, nrm, W2, root2, bias2)


# speedup vs baseline: 49.0210x; 1.5466x over previous
import functools

import jax
import jax.numpy as jnp
from jax import lax
from jax.experimental import pallas as pl
from jax.experimental.pallas import tpu as pltpu
from jax.experimental.pallas import tpu_sc as plsc

N = 10000
E = 320000
R = 4
D = 128
NRPAD = 40960
NC, NS = 2, 16
NW = NC * NS
EPT = E // NW
K = 80
NCHUNK = EPT // K
NPAD = 10240
ROWS_PT = NPAD // NS

_mesh = plsc.VectorSubcoreMesh(core_axis_name="c", subcore_axis_name="s")
_sc_params = pltpu.CompilerParams(needs_layout_passes=False)


def _mm_body(x_ref, w_ref, o_ref):
    o_ref[0] = jnp.dot(x_ref[...], w_ref[0], preferred_element_type=jnp.float32)


def _mm(x, wc):
    bm = 400
    return pl.pallas_call(
        _mm_body,
        grid=(R + 1, N // bm),
        in_specs=[
            pl.BlockSpec((bm, D), lambda r, j: (j, 0)),
            pl.BlockSpec((1, D, D), lambda r, j: (r, 0, 0)),
        ],
        out_specs=pl.BlockSpec((1, bm, D), lambda r, j: (r, j, 0)),
        out_shape=jax.ShapeDtypeStruct((R + 1, N, D), jnp.float32),
    )(x, wc)


def _c16(v):
    return jnp.full((16,), v, jnp.int32)


def _counts_body(pk_hbm, out_hbm, pk_v, cnt_v):
    cid = lax.axis_index("c")
    sid = lax.axis_index("s")
    wid = cid * NS + sid
    base = wid * EPT

    def zero_body(i, _):
        cnt_v[pl.ds(i * 16, 16)] = jnp.zeros((16,), jnp.float32)
        return 0

    lax.fori_loop(0, NRPAD // 16, zero_body, 0)

    lane_io = lax.broadcasted_iota(jnp.int32, (16,), 0)
    ones = jnp.ones((16,), jnp.float32)

    def chunk_body(j, _):
        off = base + j * K
        pltpu.sync_copy(pk_hbm.at[pl.ds(off, K)], pk_v)
        for k in range(K // 16):
            sl = pl.ds(16 * k, 16)
            cidx = lax.shift_right_logical(pk_v[sl], _c16(16))
            for lane in range(16):
                plsc.addupdate_scatter(cnt_v, [cidx], ones,
                                       mask=lane_io == lane)
        return 0

    lax.fori_loop(0, NCHUNK, chunk_body, 0)
    pltpu.sync_copy(cnt_v, out_hbm.at[wid])


@functools.partial(
    pl.kernel,
    out_type=jax.ShapeDtypeStruct((NW, NRPAD), jnp.float32),
    mesh=_mesh,
    scratch_types=[
        pltpu.VMEM((K,), jnp.int32),
        pltpu.VMEM((NRPAD,), jnp.float32),
    ],
    compiler_params=_sc_params,
)
def _sc_counts(pk_hbm, out_hbm, pk_v, cnt_v):
    _counts_body(pk_hbm, out_hbm, pk_v, cnt_v)


def _inv_body(c_ref, o_ref):
    s = jnp.sum(c_ref[...], axis=0)
    o_ref[...] = 1.0 / jnp.maximum(s, 1.0)


def _inv(counts):
    c3 = counts.reshape(NW, NRPAD // 128, 128)
    out = pl.pallas_call(
        _inv_body,
        grid=(NRPAD // (8 * 128),),
        in_specs=[pl.BlockSpec((NW, 8, 128), lambda i: (0, i, 0))],
        out_specs=pl.BlockSpec((8, 128), lambda i: (i, 0)),
        out_shape=jax.ShapeDtypeStruct((NRPAD // 128, 128), jnp.float32),
    )(c3)
    return out.reshape(NRPAD)


RECW = 3 * K


def _norm_body(pk_hbm, inv_hbm, out_hbm, pk_v, rec_v, inv_v):
    cid = lax.axis_index("c")
    sid = lax.axis_index("s")
    wid = cid * NS + sid
    base = wid * EPT

    pltpu.sync_copy(inv_hbm, inv_v)

    def chunk_body(j, _):
        off = base + j * K
        pltpu.sync_copy(pk_hbm.at[pl.ds(off, K)], pk_v)
        for k in range(K // 16):
            sl = pl.ds(16 * k, 16)
            w = pk_v[sl]
            cidx = lax.shift_right_logical(w, _c16(16))
            t = cidx & _c16(3)
            rec_v[pl.ds(16 * k, 16)] = t * N + (w & _c16(0x3FFF))
            rec_v[pl.ds(K + 16 * k, 16)] = lax.shift_right_logical(w, _c16(18))
            rec_v[pl.ds(2 * K + 16 * k, 16)] = plsc.bitcast(
                plsc.load_gather(inv_v, [cidx]), jnp.int32)
        pltpu.sync_copy(rec_v, out_hbm.at[pl.ds((wid * NCHUNK + j) * RECW,
                                                RECW)])
        return 0

    lax.fori_loop(0, NCHUNK, chunk_body, 0)


@functools.partial(
    pl.kernel,
    out_type=jax.ShapeDtypeStruct((NW * NCHUNK * RECW,), jnp.int32),
    mesh=_mesh,
    scratch_types=[
        pltpu.VMEM((K,), jnp.int32),
        pltpu.VMEM((RECW,), jnp.int32),
        pltpu.VMEM((NRPAD,), jnp.float32),
    ],
    compiler_params=_sc_params,
)
def _sc_norm(pk_hbm, inv_hbm, out_hbm, pk_v, rec_v, inv_v):
    _norm_body(pk_hbm, inv_hbm, out_hbm, pk_v, rec_v, inv_v)


NB = 4


def _msg_body(rec_hbm, tab_hbm, out_hbm,
              rec_v, dst_v, gidx_v, rows_v, rsem, gsem, ssem,
              agg_sp):
    cid = lax.axis_index("c")
    sid = lax.axis_index("s")
    wid = cid * NS + sid
    gbase = wid * NCHUNK

    def zb(i, _):
        for k in range(8):
            rows_v[0][i, pl.ds(16 * k, 16)] = jnp.zeros((16,), jnp.float32)
        return 0

    lax.fori_loop(0, K, zb, 0)
    for j in range(ROWS_PT // K):
        pltpu.sync_copy(rows_v[0], agg_sp.at[pl.ds(sid * ROWS_PT + j * K, K)])
    plsc.subcore_barrier()

    def fire_rec(j, b):
        pltpu.async_copy(rec_hbm.at[pl.ds((gbase + j) * RECW, RECW)],
                         rec_v[b], rsem[b])

    def prep_gather(j, b, first=False):
        if not first:
            pltpu.make_async_copy(rows_v[b], agg_sp.at[dst_v[b]],
                                  ssem[b]).wait()
        pltpu.make_async_copy(rec_hbm.at[pl.ds((gbase + j) * RECW, RECW)],
                              rec_v[b], rsem[b]).wait()
        for k in range(K // 16):
            sl = pl.ds(16 * k, 16)
            gidx_v[b][sl] = rec_v[b][sl]
            dst_v[b][sl] = rec_v[b][pl.ds(K + 16 * k, 16)]
        pltpu.async_copy(tab_hbm.at[gidx_v[b]], rows_v[b], gsem[b])

    def process(j, b):
        pltpu.make_async_copy(tab_hbm.at[gidx_v[b]], rows_v[b],
                              gsem[b]).wait()

        def scale_body(e, _):
            bc = plsc.bitcast(
                plsc.load_gather(rec_v[b], [jnp.broadcast_to(e + 2 * K,
                                                             (16,))]),
                jnp.float32)
            for k in range(8):
                sl = pl.ds(16 * k, 16)
                rows_v[b][e, sl] = rows_v[b][e, sl] * bc
            return 0

        lax.fori_loop(0, K, scale_body, 0)
        pltpu.make_async_copy(rows_v[b], agg_sp.at[dst_v[b]],
                              ssem[b]).start(add=True)

    fire_rec(0, 0)
    fire_rec(1, 1)
    fire_rec(2, 2)
    prep_gather(0, 0, first=True)
    prep_gather(1, 1, first=True)

    for t in range(NB):
        fire_rec(t + 3, (t + 3) % NB)
        prep_gather(t + 2, (t + 2) % NB, first=(t < 2))
        process(t, t)

    def quad(i, _):
        for t in range(NB):
            j = NB * i + t
            fire_rec(j + 3, (t + 3) % NB)
            prep_gather(j + 2, (t + 2) % NB)
            process(j, t)
        return 0

    nq = (NCHUNK - 5) // NB
    lax.fori_loop(1, nq, quad, 0)
    for j in range(NB * nq, NCHUNK):
        if j + 3 < NCHUNK:
            fire_rec(j + 3, (j + 3) % NB)
        if j + 2 < NCHUNK:
            prep_gather(j + 2, (j + 2) % NB)
        process(j, j % NB)
    for b in range(NB):
        pltpu.make_async_copy(rows_v[b], agg_sp.at[dst_v[b]], ssem[b]).wait()

    plsc.subcore_barrier()
    pltpu.sync_copy(agg_sp.at[pl.ds(sid * ROWS_PT, ROWS_PT)],
                    out_hbm.at[cid, pl.ds(sid * ROWS_PT, ROWS_PT)])


@functools.partial(
    pl.kernel,
    out_type=jax.ShapeDtypeStruct((NC, NPAD, D), jnp.float32),
    mesh=_mesh,
    scratch_types=[
        [pltpu.VMEM((RECW,), jnp.int32)] * NB,
        [pltpu.VMEM((K,), jnp.int32)] * NB,
        [pltpu.VMEM((K,), jnp.int32)] * NB,
        [pltpu.VMEM((K, D), jnp.float32)] * NB,
        [pltpu.SemaphoreType.DMA] * NB,
        [pltpu.SemaphoreType.DMA] * NB,
        [pltpu.SemaphoreType.DMA] * NB,
        pltpu.VMEM_SHARED((NPAD, D), jnp.float32),
    ],
    compiler_params=_sc_params,
)
def _sc_msg(rec_hbm, tab_hbm, out_hbm,
            rec_v, dst_v, gidx_v, rows_v, rsem, gsem, ssem,
            agg_sp):
    _msg_body(rec_hbm, tab_hbm, out_hbm,
              rec_v, dst_v, gidx_v, rows_v, rsem, gsem, ssem,
              agg_sp)


def _fin_body(a_ref, xr_ref, b_ref, o_ref):
    o_ref[...] = jnp.tanh(a_ref[0] + a_ref[1] + xr_ref[...] + b_ref[...])


def _finish(aggp, xroot, bias):
    bm = 400
    return pl.pallas_call(
        _fin_body,
        grid=(N // bm,),
        in_specs=[
            pl.BlockSpec((NC, bm, D), lambda i: (0, i, 0)),
            pl.BlockSpec((bm, D), lambda i: (i, 0)),
            pl.BlockSpec((1, D), lambda i: (0, 0)),
        ],
        out_specs=pl.BlockSpec((bm, D), lambda i: (i, 0)),
        out_shape=jax.ShapeDtypeStruct((N, D), jnp.float32),
    )(aggp, xroot, bias.reshape(1, D))


def _layer(h, rec, W, root, bias):
    wc = jnp.concatenate([W, root[None]], axis=0)
    xw = _mm(h, wc)
    table = xw.reshape((R + 1) * N, D)
    aggp = _sc_msg(rec, table)
    return _finish(aggp, xw[R], bias)


def kernel(x, edge_index, edge_attr, W1, root1, bias1, W2, root2, bias2):
    src = edge_index[0]
    dst = edge_index[1]
    pk = (dst << 18) | (edge_attr << 16) | src
    counts = _sc_counts(pk)
    inv = _inv(counts)
    rec = _sc_norm(pk, inv)
    h = _layer(x, rec, W1, root1, bias1)
    return _layer(h, rec, W2, root2, bias2)


# speedup vs baseline: 52.0054x; 1.0609x over previous
import functools

import jax
import jax.numpy as jnp
from jax import lax
from jax.experimental import pallas as pl
from jax.experimental.pallas import tpu as pltpu
from jax.experimental.pallas import tpu_sc as plsc

N = 10000
E = 320000
R = 4
D = 128
NRPAD = 40960
NC, NS = 2, 16
NW = NC * NS
EPT = E // NW
K = 80
NCHUNK = EPT // K
NPAD = 10240
ROWS_PT = NPAD // NS

_mesh = plsc.VectorSubcoreMesh(core_axis_name="c", subcore_axis_name="s")
_sc_params = pltpu.CompilerParams(needs_layout_passes=False)


def _mm_body(x_ref, w_ref, o_ref):
    o_ref[0] = jnp.dot(x_ref[...], w_ref[0], preferred_element_type=jnp.float32)


def _mm(x, wc):
    bm = 400
    return pl.pallas_call(
        _mm_body,
        grid=(R + 1, N // bm),
        in_specs=[
            pl.BlockSpec((bm, D), lambda r, j: (j, 0)),
            pl.BlockSpec((1, D, D), lambda r, j: (r, 0, 0)),
        ],
        out_specs=pl.BlockSpec((1, bm, D), lambda r, j: (r, j, 0)),
        out_shape=jax.ShapeDtypeStruct((R + 1, N, D), jnp.float32),
    )(x, wc)


def _c16(v):
    return jnp.full((16,), v, jnp.int32)


def _counts_body(pk_hbm, out_hbm, pk_v, cnt_v, lsem):
    cid = lax.axis_index("c")
    sid = lax.axis_index("s")
    wid = cid * NS + sid
    base = wid * EPT

    def zero_body(i, _):
        cnt_v[pl.ds(i * 16, 16)] = jnp.zeros((16,), jnp.float32)
        return 0

    lax.fori_loop(0, NRPAD // 16, zero_body, 0)

    lane_io = lax.broadcasted_iota(jnp.int32, (16,), 0)
    ones = jnp.ones((16,), jnp.float32)
    masks = [lane_io == _c16(lane) for lane in range(16)]

    def fire(j, b):
        pltpu.async_copy(pk_hbm.at[pl.ds(base + j * K, K)], pk_v[b], lsem[b])

    def count(j, b):
        pltpu.make_async_copy(pk_hbm.at[pl.ds(base + j * K, K)], pk_v[b],
                              lsem[b]).wait()
        for k in range(K // 16):
            sl = pl.ds(16 * k, 16)
            cidx = lax.shift_right_logical(pk_v[b][sl], _c16(16))
            for lane in range(16):
                plsc.addupdate_scatter(cnt_v, [cidx], ones, mask=masks[lane])

    fire(0, 0)

    def pair(i, _):
        j = 2 * i
        fire(j + 1, 1)
        count(j, 0)
        fire(j + 2, 0)
        count(j + 1, 1)
        return 0

    lax.fori_loop(0, (NCHUNK - 1) // 2, pair, 0)
    count(NCHUNK - 1, 0)
    pltpu.sync_copy(cnt_v, out_hbm.at[wid])


@functools.partial(
    pl.kernel,
    out_type=jax.ShapeDtypeStruct((NW, NRPAD), jnp.float32),
    mesh=_mesh,
    scratch_types=[
        [pltpu.VMEM((K,), jnp.int32)] * 2,
        pltpu.VMEM((NRPAD,), jnp.float32),
        [pltpu.SemaphoreType.DMA] * 2,
    ],
    compiler_params=_sc_params,
)
def _sc_counts(pk_hbm, out_hbm, pk_v, cnt_v, lsem):
    _counts_body(pk_hbm, out_hbm, pk_v, cnt_v, lsem)


def _inv_body(c_ref, o_ref):
    s = jnp.sum(c_ref[...], axis=0)
    o_ref[...] = 1.0 / jnp.maximum(s, 1.0)


def _inv(counts):
    c3 = counts.reshape(NW, NRPAD // 128, 128)
    out = pl.pallas_call(
        _inv_body,
        grid=(NRPAD // (8 * 128),),
        in_specs=[pl.BlockSpec((NW, 8, 128), lambda i: (0, i, 0))],
        out_specs=pl.BlockSpec((8, 128), lambda i: (i, 0)),
        out_shape=jax.ShapeDtypeStruct((NRPAD // 128, 128), jnp.float32),
    )(c3)
    return out.reshape(NRPAD)


RECW = 3 * K


def _norm_body(pk_hbm, inv_hbm, out_hbm, pk_v, rec_v, inv_v, lsem, wsem):
    cid = lax.axis_index("c")
    sid = lax.axis_index("s")
    wid = cid * NS + sid
    base = wid * EPT

    pltpu.sync_copy(inv_hbm, inv_v)

    def fire(j, b):
        pltpu.async_copy(pk_hbm.at[pl.ds(base + j * K, K)], pk_v[b], lsem[b])

    def build(j, b, first=False):
        pltpu.make_async_copy(pk_hbm.at[pl.ds(base + j * K, K)], pk_v[b],
                              lsem[b]).wait()
        if not first:
            pltpu.make_async_copy(rec_v[b], out_hbm.at[pl.ds(0, RECW)],
                                  wsem[b]).wait()
        for k in range(K // 16):
            sl = pl.ds(16 * k, 16)
            w = pk_v[b][sl]
            cidx = lax.shift_right_logical(w, _c16(16))
            t = cidx & _c16(3)
            rec_v[b][pl.ds(16 * k, 16)] = t * N + (w & _c16(0x3FFF))
            rec_v[b][pl.ds(K + 16 * k, 16)] = lax.shift_right_logical(
                w, _c16(18))
            rec_v[b][pl.ds(2 * K + 16 * k, 16)] = plsc.bitcast(
                plsc.load_gather(inv_v, [cidx]), jnp.int32)
        pltpu.make_async_copy(
            rec_v[b],
            out_hbm.at[pl.ds((wid * NCHUNK + j) * RECW, RECW)],
            wsem[b]).start()

    fire(0, 0)
    fire(1, 1)
    build(0, 0, first=True)
    fire(2, 0)
    build(1, 1, first=True)

    def pair(i, _):
        j = 2 * i
        fire(j + 1, 1)
        build(j, 0)
        fire(j + 2, 0)
        build(j + 1, 1)
        return 0

    lax.fori_loop(1, (NCHUNK - 1) // 2, pair, 0)
    build(NCHUNK - 1, 0)
    for b in range(2):
        pltpu.make_async_copy(rec_v[b], out_hbm.at[pl.ds(0, RECW)],
                              wsem[b]).wait()


@functools.partial(
    pl.kernel,
    out_type=jax.ShapeDtypeStruct((NW * NCHUNK * RECW,), jnp.int32),
    mesh=_mesh,
    scratch_types=[
        [pltpu.VMEM((K,), jnp.int32)] * 2,
        [pltpu.VMEM((RECW,), jnp.int32)] * 2,
        pltpu.VMEM((NRPAD,), jnp.float32),
        [pltpu.SemaphoreType.DMA] * 2,
        [pltpu.SemaphoreType.DMA] * 2,
    ],
    compiler_params=_sc_params,
)
def _sc_norm(pk_hbm, inv_hbm, out_hbm, pk_v, rec_v, inv_v, lsem, wsem):
    _norm_body(pk_hbm, inv_hbm, out_hbm, pk_v, rec_v, inv_v, lsem, wsem)


NB = 4


def _msg_body(rec_hbm, tab_hbm, out_hbm,
              rec_v, dst_v, gidx_v, rows_v, rsem, gsem, ssem,
              agg_sp):
    cid = lax.axis_index("c")
    sid = lax.axis_index("s")
    wid = cid * NS + sid
    gbase = wid * NCHUNK

    def zb(i, _):
        for k in range(8):
            rows_v[0][i, pl.ds(16 * k, 16)] = jnp.zeros((16,), jnp.float32)
        return 0

    lax.fori_loop(0, K, zb, 0)
    for j in range(ROWS_PT // K):
        pltpu.sync_copy(rows_v[0], agg_sp.at[pl.ds(sid * ROWS_PT + j * K, K)])
    plsc.subcore_barrier()

    def fire_rec(j, b):
        pltpu.async_copy(rec_hbm.at[pl.ds((gbase + j) * RECW, RECW)],
                         rec_v[b], rsem[b])

    def prep_gather(j, b, first=False):
        if not first:
            pltpu.make_async_copy(rows_v[b], agg_sp.at[dst_v[b]],
                                  ssem[b]).wait()
        pltpu.make_async_copy(rec_hbm.at[pl.ds((gbase + j) * RECW, RECW)],
                              rec_v[b], rsem[b]).wait()
        for k in range(K // 16):
            sl = pl.ds(16 * k, 16)
            gidx_v[b][sl] = rec_v[b][sl]
            dst_v[b][sl] = rec_v[b][pl.ds(K + 16 * k, 16)]
        pltpu.async_copy(tab_hbm.at[gidx_v[b]], rows_v[b], gsem[b])

    def process(j, b):
        pltpu.make_async_copy(tab_hbm.at[gidx_v[b]], rows_v[b],
                              gsem[b]).wait()

        def scale_body(e, _):
            bc = plsc.bitcast(
                plsc.load_gather(rec_v[b], [jnp.broadcast_to(e + 2 * K,
                                                             (16,))]),
                jnp.float32)
            for k in range(8):
                sl = pl.ds(16 * k, 16)
                rows_v[b][e, sl] = rows_v[b][e, sl] * bc
            return 0

        lax.fori_loop(0, K, scale_body, 0, unroll=4)
        pltpu.make_async_copy(rows_v[b], agg_sp.at[dst_v[b]],
                              ssem[b]).start(add=True)

    fire_rec(0, 0)
    fire_rec(1, 1)
    fire_rec(2, 2)
    prep_gather(0, 0, first=True)
    prep_gather(1, 1, first=True)

    for t in range(NB):
        fire_rec(t + 3, (t + 3) % NB)
        prep_gather(t + 2, (t + 2) % NB, first=(t < 2))
        process(t, t)

    def quad(i, _):
        for t in range(NB):
            j = NB * i + t
            fire_rec(j + 3, (t + 3) % NB)
            prep_gather(j + 2, (t + 2) % NB)
            process(j, t)
        return 0

    nq = (NCHUNK - 5) // NB
    lax.fori_loop(1, nq, quad, 0)
    for j in range(NB * nq, NCHUNK):
        if j + 3 < NCHUNK:
            fire_rec(j + 3, (j + 3) % NB)
        if j + 2 < NCHUNK:
            prep_gather(j + 2, (j + 2) % NB)
        process(j, j % NB)
    for b in range(NB):
        pltpu.make_async_copy(rows_v[b], agg_sp.at[dst_v[b]], ssem[b]).wait()

    plsc.subcore_barrier()
    pltpu.sync_copy(agg_sp.at[pl.ds(sid * ROWS_PT, ROWS_PT)],
                    out_hbm.at[cid, pl.ds(sid * ROWS_PT, ROWS_PT)])


@functools.partial(
    pl.kernel,
    out_type=jax.ShapeDtypeStruct((NC, NPAD, D), jnp.float32),
    mesh=_mesh,
    scratch_types=[
        [pltpu.VMEM((RECW,), jnp.int32)] * NB,
        [pltpu.VMEM((K,), jnp.int32)] * NB,
        [pltpu.VMEM((K,), jnp.int32)] * NB,
        [pltpu.VMEM((K, D), jnp.float32)] * NB,
        [pltpu.SemaphoreType.DMA] * NB,
        [pltpu.SemaphoreType.DMA] * NB,
        [pltpu.SemaphoreType.DMA] * NB,
        pltpu.VMEM_SHARED((NPAD, D), jnp.float32),
    ],
    compiler_params=_sc_params,
)
def _sc_msg(rec_hbm, tab_hbm, out_hbm,
            rec_v, dst_v, gidx_v, rows_v, rsem, gsem, ssem,
            agg_sp):
    _msg_body(rec_hbm, tab_hbm, out_hbm,
              rec_v, dst_v, gidx_v, rows_v, rsem, gsem, ssem,
              agg_sp)


def _fin_body(a_ref, xr_ref, b_ref, o_ref):
    o_ref[...] = jnp.tanh(a_ref[0] + a_ref[1] + xr_ref[...] + b_ref[...])


def _finish(aggp, xroot, bias):
    bm = 400
    return pl.pallas_call(
        _fin_body,
        grid=(N // bm,),
        in_specs=[
            pl.BlockSpec((NC, bm, D), lambda i: (0, i, 0)),
            pl.BlockSpec((bm, D), lambda i: (i, 0)),
            pl.BlockSpec((1, D), lambda i: (0, 0)),
        ],
        out_specs=pl.BlockSpec((bm, D), lambda i: (i, 0)),
        out_shape=jax.ShapeDtypeStruct((N, D), jnp.float32),
    )(aggp, xroot, bias.reshape(1, D))


def _layer(h, rec, W, root, bias):
    wc = jnp.concatenate([W, root[None]], axis=0)
    xw = _mm(h, wc)
    table = xw.reshape((R + 1) * N, D)
    aggp = _sc_msg(rec, table)
    return _finish(aggp, xw[R], bias)


def kernel(x, edge_index, edge_attr, W1, root1, bias1, W2, root2, bias2):
    src = edge_index[0]
    dst = edge_index[1]
    pk = (dst << 18) | (edge_attr << 16) | src
    counts = _sc_counts(pk)
    inv = _inv(counts)
    rec = _sc_norm(pk, inv)
    h = _layer(x, rec, W1, root1, bias1)
    return _layer(h, rec, W2, root2, bias2)


# speedup vs baseline: 64.0468x; 1.2315x over previous
import functools

import jax
import jax.numpy as jnp
from jax import lax
from jax.experimental import pallas as pl
from jax.experimental.pallas import tpu as pltpu
from jax.experimental.pallas import tpu_sc as plsc

N = 10000
E = 320000
R = 4
D = 128
NRPAD = 40960
NC, NS = 2, 16
NW = NC * NS
EPT = E // NW
K = 80
NCHUNK = EPT // K
NPAD = 10240
ROWS_PT = NPAD // NS

_mesh = plsc.VectorSubcoreMesh(core_axis_name="c", subcore_axis_name="s")
_sc_params = pltpu.CompilerParams(needs_layout_passes=False)


def _mm_body(x_ref, w_ref, o_ref):
    o_ref[0] = jnp.dot(x_ref[...], w_ref[0], preferred_element_type=jnp.float32)


def _mm(x, wc):
    bm = 2000
    return pl.pallas_call(
        _mm_body,
        grid=(R + 1, N // bm),
        in_specs=[
            pl.BlockSpec((bm, D), lambda r, j: (j, 0)),
            pl.BlockSpec((1, D, D), lambda r, j: (r, 0, 0)),
        ],
        out_specs=pl.BlockSpec((1, bm, D), lambda r, j: (r, j, 0)),
        out_shape=jax.ShapeDtypeStruct((R + 1, N, D), jnp.float32),
    )(x, wc)


def _c16(v):
    return jnp.full((16,), v, jnp.int32)


def _counts_body(pk_hbm, out_hbm, pk_v, cnt_v, lsem):
    cid = lax.axis_index("c")
    sid = lax.axis_index("s")
    wid = cid * NS + sid
    base = wid * EPT

    def zero_body(i, _):
        cnt_v[pl.ds(i * 16, 16)] = jnp.zeros((16,), jnp.float32)
        return 0

    lax.fori_loop(0, NRPAD // 16, zero_body, 0)

    lane_io = lax.broadcasted_iota(jnp.int32, (16,), 0)
    ones = jnp.ones((16,), jnp.float32)
    masks = [lane_io == _c16(lane) for lane in range(16)]

    def fire(j, b):
        pltpu.async_copy(pk_hbm.at[pl.ds(base + j * K, K)], pk_v[b], lsem[b])

    def count(j, b):
        pltpu.make_async_copy(pk_hbm.at[pl.ds(base + j * K, K)], pk_v[b],
                              lsem[b]).wait()
        for k in range(K // 16):
            sl = pl.ds(16 * k, 16)
            cidx = lax.shift_right_logical(pk_v[b][sl], _c16(16))
            for lane in range(16):
                plsc.addupdate_scatter(cnt_v, [cidx], ones, mask=masks[lane])

    fire(0, 0)

    def pair(i, _):
        j = 2 * i
        fire(j + 1, 1)
        count(j, 0)
        fire(j + 2, 0)
        count(j + 1, 1)
        return 0

    lax.fori_loop(0, (NCHUNK - 1) // 2, pair, 0)
    count(NCHUNK - 1, 0)
    pltpu.sync_copy(cnt_v, out_hbm.at[wid])


@functools.partial(
    pl.kernel,
    out_type=jax.ShapeDtypeStruct((NW, NRPAD), jnp.float32),
    mesh=_mesh,
    scratch_types=[
        [pltpu.VMEM((K,), jnp.int32)] * 2,
        pltpu.VMEM((NRPAD,), jnp.float32),
        [pltpu.SemaphoreType.DMA] * 2,
    ],
    compiler_params=_sc_params,
)
def _sc_counts(pk_hbm, out_hbm, pk_v, cnt_v, lsem):
    _counts_body(pk_hbm, out_hbm, pk_v, cnt_v, lsem)


def _inv_body(c_ref, o_ref):
    s = jnp.sum(c_ref[...], axis=0)
    o_ref[...] = 1.0 / jnp.maximum(s, 1.0)


def _inv(counts):
    c3 = counts.reshape(NW, NRPAD // 128, 128)
    out = pl.pallas_call(
        _inv_body,
        grid=(NRPAD // (40 * 128),),
        in_specs=[pl.BlockSpec((NW, 40, 128), lambda i: (0, i, 0))],
        out_specs=pl.BlockSpec((40, 128), lambda i: (i, 0)),
        out_shape=jax.ShapeDtypeStruct((NRPAD // 128, 128), jnp.float32),
    )(c3)
    return out.reshape(NRPAD)


RECW = 3 * K


def _norm_body(pk_hbm, inv_hbm, out_hbm, pk_v, rec_v, inv_v, lsem, wsem):
    cid = lax.axis_index("c")
    sid = lax.axis_index("s")
    wid = cid * NS + sid
    base = wid * EPT

    pltpu.sync_copy(inv_hbm, inv_v)

    def fire(j, b):
        pltpu.async_copy(pk_hbm.at[pl.ds(base + j * K, K)], pk_v[b], lsem[b])

    def build(j, b, first=False):
        pltpu.make_async_copy(pk_hbm.at[pl.ds(base + j * K, K)], pk_v[b],
                              lsem[b]).wait()
        if not first:
            pltpu.make_async_copy(rec_v[b], out_hbm.at[pl.ds(0, RECW)],
                                  wsem[b]).wait()
        for k in range(K // 16):
            sl = pl.ds(16 * k, 16)
            w = pk_v[b][sl]
            cidx = lax.shift_right_logical(w, _c16(16))
            t = cidx & _c16(3)
            rec_v[b][pl.ds(16 * k, 16)] = t * N + (w & _c16(0x3FFF))
            rec_v[b][pl.ds(K + 16 * k, 16)] = lax.shift_right_logical(
                w, _c16(18))
            rec_v[b][pl.ds(2 * K + 16 * k, 16)] = plsc.bitcast(
                plsc.load_gather(inv_v, [cidx]), jnp.int32)
        pltpu.make_async_copy(
            rec_v[b],
            out_hbm.at[pl.ds((wid * NCHUNK + j) * RECW, RECW)],
            wsem[b]).start()

    fire(0, 0)
    fire(1, 1)
    build(0, 0, first=True)
    fire(2, 0)
    build(1, 1, first=True)

    def pair(i, _):
        j = 2 * i
        fire(j + 1, 1)
        build(j, 0)
        fire(j + 2, 0)
        build(j + 1, 1)
        return 0

    lax.fori_loop(1, (NCHUNK - 1) // 2, pair, 0)
    build(NCHUNK - 1, 0)
    for b in range(2):
        pltpu.make_async_copy(rec_v[b], out_hbm.at[pl.ds(0, RECW)],
                              wsem[b]).wait()


@functools.partial(
    pl.kernel,
    out_type=jax.ShapeDtypeStruct((NW * NCHUNK * RECW,), jnp.int32),
    mesh=_mesh,
    scratch_types=[
        [pltpu.VMEM((K,), jnp.int32)] * 2,
        [pltpu.VMEM((RECW,), jnp.int32)] * 2,
        pltpu.VMEM((NRPAD,), jnp.float32),
        [pltpu.SemaphoreType.DMA] * 2,
        [pltpu.SemaphoreType.DMA] * 2,
    ],
    compiler_params=_sc_params,
)
def _sc_norm(pk_hbm, inv_hbm, out_hbm, pk_v, rec_v, inv_v, lsem, wsem):
    _norm_body(pk_hbm, inv_hbm, out_hbm, pk_v, rec_v, inv_v, lsem, wsem)


NB = 4


def _msg_body(rec_hbm, tab_hbm, out_hbm,
              rec_v, dst_v, gidx_v, rows_v, rsem, gsem, ssem,
              agg_sp):
    cid = lax.axis_index("c")
    sid = lax.axis_index("s")
    wid = cid * NS + sid
    gbase = wid * NCHUNK

    def zb(i, _):
        for k in range(8):
            rows_v[0][i, pl.ds(16 * k, 16)] = jnp.zeros((16,), jnp.float32)
        return 0

    lax.fori_loop(0, K, zb, 0)
    for j in range(ROWS_PT // K):
        pltpu.sync_copy(rows_v[0], agg_sp.at[pl.ds(sid * ROWS_PT + j * K, K)])
    plsc.subcore_barrier()

    def fire_rec(j, b):
        pltpu.async_copy(rec_hbm.at[pl.ds((gbase + j) * RECW, RECW)],
                         rec_v[b], rsem[b])

    def prep_gather(j, b, first=False):
        if not first:
            pltpu.make_async_copy(rows_v[b], agg_sp.at[dst_v[b]],
                                  ssem[b]).wait()
        pltpu.make_async_copy(rec_hbm.at[pl.ds((gbase + j) * RECW, RECW)],
                              rec_v[b], rsem[b]).wait()
        for k in range(K // 16):
            sl = pl.ds(16 * k, 16)
            gidx_v[b][sl] = rec_v[b][sl]
            dst_v[b][sl] = rec_v[b][pl.ds(K + 16 * k, 16)]
        pltpu.async_copy(tab_hbm.at[gidx_v[b]], rows_v[b], gsem[b])

    def process(j, b):
        pltpu.make_async_copy(tab_hbm.at[gidx_v[b]], rows_v[b],
                              gsem[b]).wait()

        def scale_body(e, _):
            bc = plsc.bitcast(
                plsc.load_gather(rec_v[b], [jnp.broadcast_to(e + 2 * K,
                                                             (16,))]),
                jnp.float32)
            for k in range(8):
                sl = pl.ds(16 * k, 16)
                rows_v[b][e, sl] = rows_v[b][e, sl] * bc
            return 0

        lax.fori_loop(0, K, scale_body, 0, unroll=4)
        pltpu.make_async_copy(rows_v[b], agg_sp.at[dst_v[b]],
                              ssem[b]).start(add=True)

    fire_rec(0, 0)
    fire_rec(1, 1)
    fire_rec(2, 2)
    prep_gather(0, 0, first=True)
    prep_gather(1, 1, first=True)

    for t in range(NB):
        fire_rec(t + 3, (t + 3) % NB)
        prep_gather(t + 2, (t + 2) % NB, first=(t < 2))
        process(t, t)

    def quad(i, _):
        for t in range(NB):
            j = NB * i + t
            fire_rec(j + 3, (t + 3) % NB)
            prep_gather(j + 2, (t + 2) % NB)
            process(j, t)
        return 0

    nq = (NCHUNK - 5) // NB
    lax.fori_loop(1, nq, quad, 0)
    for j in range(NB * nq, NCHUNK):
        if j + 3 < NCHUNK:
            fire_rec(j + 3, (j + 3) % NB)
        if j + 2 < NCHUNK:
            prep_gather(j + 2, (j + 2) % NB)
        process(j, j % NB)
    for b in range(NB):
        pltpu.make_async_copy(rows_v[b], agg_sp.at[dst_v[b]], ssem[b]).wait()

    plsc.subcore_barrier()
    pltpu.sync_copy(agg_sp.at[pl.ds(sid * ROWS_PT, ROWS_PT)],
                    out_hbm.at[cid, pl.ds(sid * ROWS_PT, ROWS_PT)])


@functools.partial(
    pl.kernel,
    out_type=jax.ShapeDtypeStruct((NC, NPAD, D), jnp.float32),
    mesh=_mesh,
    scratch_types=[
        [pltpu.VMEM((RECW,), jnp.int32)] * NB,
        [pltpu.VMEM((K,), jnp.int32)] * NB,
        [pltpu.VMEM((K,), jnp.int32)] * NB,
        [pltpu.VMEM((K, D), jnp.float32)] * NB,
        [pltpu.SemaphoreType.DMA] * NB,
        [pltpu.SemaphoreType.DMA] * NB,
        [pltpu.SemaphoreType.DMA] * NB,
        pltpu.VMEM_SHARED((NPAD, D), jnp.float32),
    ],
    compiler_params=_sc_params,
)
def _sc_msg(rec_hbm, tab_hbm, out_hbm,
            rec_v, dst_v, gidx_v, rows_v, rsem, gsem, ssem,
            agg_sp):
    _msg_body(rec_hbm, tab_hbm, out_hbm,
              rec_v, dst_v, gidx_v, rows_v, rsem, gsem, ssem,
              agg_sp)


def _fin_body(a_ref, xr_ref, b_ref, o_ref):
    o_ref[...] = jnp.tanh(a_ref[0] + a_ref[1] + xr_ref[...] + b_ref[...])


def _finish(aggp, xroot, bias):
    bm = 400
    return pl.pallas_call(
        _fin_body,
        grid=(N // bm,),
        in_specs=[
            pl.BlockSpec((NC, bm, D), lambda i: (0, i, 0)),
            pl.BlockSpec((bm, D), lambda i: (i, 0)),
            pl.BlockSpec((1, D), lambda i: (0, 0)),
        ],
        out_specs=pl.BlockSpec((bm, D), lambda i: (i, 0)),
        out_shape=jax.ShapeDtypeStruct((N, D), jnp.float32),
    )(aggp, xroot, bias.reshape(1, D))


def _layer(h, rec, W, root, bias):
    wc = jnp.concatenate([W, root[None]], axis=0)
    xw = _mm(h, wc)
    table = xw.reshape((R + 1) * N, D)
    aggp = _sc_msg(rec, table)
    return _finish(aggp, xw[R], bias)


def kernel(x, edge_index, edge_attr, W1, root1, bias1, W2, root2, bias2):
    src = edge_index[0]
    dst = edge_index[1]
    pk = (dst << 18) | (edge_attr << 16) | src
    counts = _sc_counts(pk)
    inv = _inv(counts)
    rec = _sc_norm(pk, inv)
    h = _layer(x, rec, W1, root1, bias1)
    return _layer(h, rec, W2, root2, bias2)


# speedup vs baseline: 67.1234x; 1.0480x over previous
import functools

import jax
import jax.numpy as jnp
from jax import lax
from jax.experimental import pallas as pl
from jax.experimental.pallas import tpu as pltpu
from jax.experimental.pallas import tpu_sc as plsc

N = 10000
E = 320000
R = 4
D = 128
NRPAD = 40960
NC, NS = 2, 16
NW = NC * NS
EPT = E // NW
K = 80
NCHUNK = EPT // K
NPAD = 10240
ROWS_PT = NPAD // NS

_mesh = plsc.VectorSubcoreMesh(core_axis_name="c", subcore_axis_name="s")
_sc_params = pltpu.CompilerParams(needs_layout_passes=False)


def _mm_body(x_ref, w_ref, o_ref):
    o_ref[0] = jnp.dot(x_ref[...], w_ref[0], preferred_element_type=jnp.float32)


def _mm(x, wc):
    bm = 2000
    return pl.pallas_call(
        _mm_body,
        grid=(N // bm, R + 1),
        in_specs=[
            pl.BlockSpec((bm, D), lambda j, r: (j, 0)),
            pl.BlockSpec((1, D, D), lambda j, r: (r, 0, 0)),
        ],
        out_specs=pl.BlockSpec((1, bm, D), lambda j, r: (r, j, 0)),
        out_shape=jax.ShapeDtypeStruct((R + 1, N, D), jnp.float32),
    )(x, wc)


def _pack_body(src_ref, dst_ref, ea_ref, o_ref):
    o_ref[...] = ((dst_ref[...] << 18) | (ea_ref[...] << 16) | src_ref[...])


def _pack(edge_index, edge_attr):
    return pl.pallas_call(
        _pack_body,
        out_shape=jax.ShapeDtypeStruct((E,), jnp.int32),
    )(edge_index[0], edge_index[1], edge_attr)


def _c16(v):
    return jnp.full((16,), v, jnp.int32)


def _counts_body(pk_hbm, out_hbm, pk_v, cnt_v, lsem):
    cid = lax.axis_index("c")
    sid = lax.axis_index("s")
    wid = cid * NS + sid
    base = wid * EPT

    def zero_body(i, _):
        cnt_v[pl.ds(i * 16, 16)] = jnp.zeros((16,), jnp.float32)
        return 0

    lax.fori_loop(0, NRPAD // 16, zero_body, 0)

    lane_io = lax.broadcasted_iota(jnp.int32, (16,), 0)
    ones = jnp.ones((16,), jnp.float32)
    masks = [lane_io == _c16(lane) for lane in range(16)]

    def fire(j, b):
        pltpu.async_copy(pk_hbm.at[pl.ds(base + j * K, K)], pk_v[b], lsem[b])

    def count(j, b):
        pltpu.make_async_copy(pk_hbm.at[pl.ds(base + j * K, K)], pk_v[b],
                              lsem[b]).wait()
        for k in range(K // 16):
            sl = pl.ds(16 * k, 16)
            cidx = lax.shift_right_logical(pk_v[b][sl], _c16(16))
            for lane in range(16):
                plsc.addupdate_scatter(cnt_v, [cidx], ones, mask=masks[lane])

    fire(0, 0)

    def pair(i, _):
        j = 2 * i
        fire(j + 1, 1)
        count(j, 0)
        fire(j + 2, 0)
        count(j + 1, 1)
        return 0

    lax.fori_loop(0, (NCHUNK - 1) // 2, pair, 0)
    count(NCHUNK - 1, 0)
    pltpu.sync_copy(cnt_v, out_hbm.at[wid])


@functools.partial(
    pl.kernel,
    out_type=jax.ShapeDtypeStruct((NW, NRPAD), jnp.float32),
    mesh=_mesh,
    scratch_types=[
        [pltpu.VMEM((K,), jnp.int32)] * 2,
        pltpu.VMEM((NRPAD,), jnp.float32),
        [pltpu.SemaphoreType.DMA] * 2,
    ],
    compiler_params=_sc_params,
)
def _sc_counts(pk_hbm, out_hbm, pk_v, cnt_v, lsem):
    _counts_body(pk_hbm, out_hbm, pk_v, cnt_v, lsem)


def _inv_body(c_ref, o_ref):
    s = jnp.sum(c_ref[...], axis=0)
    o_ref[...] = 1.0 / jnp.maximum(s, 1.0)


def _inv(counts):
    c3 = counts.reshape(NW, NRPAD // 128, 128)
    out = pl.pallas_call(
        _inv_body,
        grid=(NRPAD // (40 * 128),),
        in_specs=[pl.BlockSpec((NW, 40, 128), lambda i: (0, i, 0))],
        out_specs=pl.BlockSpec((40, 128), lambda i: (i, 0)),
        out_shape=jax.ShapeDtypeStruct((NRPAD // 128, 128), jnp.float32),
    )(c3)
    return out.reshape(NRPAD)


RECW = 3 * K


def _norm_body(pk_hbm, inv_hbm, out_hbm, pk_v, rec_v, inv_v, lsem, wsem):
    cid = lax.axis_index("c")
    sid = lax.axis_index("s")
    wid = cid * NS + sid
    base = wid * EPT

    pltpu.sync_copy(inv_hbm, inv_v)

    def fire(j, b):
        pltpu.async_copy(pk_hbm.at[pl.ds(base + j * K, K)], pk_v[b], lsem[b])

    def build(j, b, first=False):
        pltpu.make_async_copy(pk_hbm.at[pl.ds(base + j * K, K)], pk_v[b],
                              lsem[b]).wait()
        if not first:
            pltpu.make_async_copy(rec_v[b], out_hbm.at[pl.ds(0, RECW)],
                                  wsem[b]).wait()
        for k in range(K // 16):
            sl = pl.ds(16 * k, 16)
            w = pk_v[b][sl]
            cidx = lax.shift_right_logical(w, _c16(16))
            t = cidx & _c16(3)
            rec_v[b][pl.ds(16 * k, 16)] = t * N + (w & _c16(0x3FFF))
            rec_v[b][pl.ds(K + 16 * k, 16)] = lax.shift_right_logical(
                w, _c16(18))
            rec_v[b][pl.ds(2 * K + 16 * k, 16)] = plsc.bitcast(
                plsc.load_gather(inv_v, [cidx]), jnp.int32)
        pltpu.make_async_copy(
            rec_v[b],
            out_hbm.at[pl.ds((wid * NCHUNK + j) * RECW, RECW)],
            wsem[b]).start()

    fire(0, 0)
    fire(1, 1)
    build(0, 0, first=True)
    fire(2, 0)
    build(1, 1, first=True)

    def pair(i, _):
        j = 2 * i
        fire(j + 1, 1)
        build(j, 0)
        fire(j + 2, 0)
        build(j + 1, 1)
        return 0

    lax.fori_loop(1, (NCHUNK - 1) // 2, pair, 0)
    build(NCHUNK - 1, 0)
    for b in range(2):
        pltpu.make_async_copy(rec_v[b], out_hbm.at[pl.ds(0, RECW)],
                              wsem[b]).wait()


@functools.partial(
    pl.kernel,
    out_type=jax.ShapeDtypeStruct((NW * NCHUNK * RECW,), jnp.int32),
    mesh=_mesh,
    scratch_types=[
        [pltpu.VMEM((K,), jnp.int32)] * 2,
        [pltpu.VMEM((RECW,), jnp.int32)] * 2,
        pltpu.VMEM((NRPAD,), jnp.float32),
        [pltpu.SemaphoreType.DMA] * 2,
        [pltpu.SemaphoreType.DMA] * 2,
    ],
    compiler_params=_sc_params,
)
def _sc_norm(pk_hbm, inv_hbm, out_hbm, pk_v, rec_v, inv_v, lsem, wsem):
    _norm_body(pk_hbm, inv_hbm, out_hbm, pk_v, rec_v, inv_v, lsem, wsem)


NB = 4


def _msg_body(rec_hbm, tab_hbm, out_hbm,
              rec_v, dst_v, gidx_v, rows_v, rsem, gsem, ssem,
              agg_sp):
    cid = lax.axis_index("c")
    sid = lax.axis_index("s")
    wid = cid * NS + sid
    gbase = wid * NCHUNK

    def zb(i, _):
        for k in range(8):
            rows_v[0][i, pl.ds(16 * k, 16)] = jnp.zeros((16,), jnp.float32)
        return 0

    lax.fori_loop(0, K, zb, 0)
    for j in range(ROWS_PT // K):
        pltpu.sync_copy(rows_v[0], agg_sp.at[pl.ds(sid * ROWS_PT + j * K, K)])
    plsc.subcore_barrier()

    def fire_rec(j, b):
        pltpu.async_copy(rec_hbm.at[pl.ds((gbase + j) * RECW, RECW)],
                         rec_v[b], rsem[b])

    def prep_gather(j, b, first=False):
        if not first:
            pltpu.make_async_copy(rows_v[b], agg_sp.at[dst_v[b]],
                                  ssem[b]).wait()
        pltpu.make_async_copy(rec_hbm.at[pl.ds((gbase + j) * RECW, RECW)],
                              rec_v[b], rsem[b]).wait()
        for k in range(K // 16):
            sl = pl.ds(16 * k, 16)
            gidx_v[b][sl] = rec_v[b][sl]
            dst_v[b][sl] = rec_v[b][pl.ds(K + 16 * k, 16)]
        pltpu.async_copy(tab_hbm.at[gidx_v[b]], rows_v[b], gsem[b])

    def process(j, b):
        pltpu.make_async_copy(tab_hbm.at[gidx_v[b]], rows_v[b],
                              gsem[b]).wait()

        def scale_body(e, _):
            bc = plsc.bitcast(
                plsc.load_gather(rec_v[b], [jnp.broadcast_to(e + 2 * K,
                                                             (16,))]),
                jnp.float32)
            for k in range(8):
                sl = pl.ds(16 * k, 16)
                rows_v[b][e, sl] = rows_v[b][e, sl] * bc
            return 0

        lax.fori_loop(0, K, scale_body, 0, unroll=4)
        pltpu.make_async_copy(rows_v[b], agg_sp.at[dst_v[b]],
                              ssem[b]).start(add=True)

    fire_rec(0, 0)
    fire_rec(1, 1)
    fire_rec(2, 2)
    prep_gather(0, 0, first=True)
    prep_gather(1, 1, first=True)

    for t in range(NB):
        fire_rec(t + 3, (t + 3) % NB)
        prep_gather(t + 2, (t + 2) % NB, first=(t < 2))
        process(t, t)

    def quad(i, _):
        for t in range(NB):
            j = NB * i + t
            fire_rec(j + 3, (t + 3) % NB)
            prep_gather(j + 2, (t + 2) % NB)
            process(j, t)
        return 0

    nq = (NCHUNK - 5) // NB
    lax.fori_loop(1, nq, quad, 0)
    for j in range(NB * nq, NCHUNK):
        if j + 3 < NCHUNK:
            fire_rec(j + 3, (j + 3) % NB)
        if j + 2 < NCHUNK:
            prep_gather(j + 2, (j + 2) % NB)
        process(j, j % NB)
    for b in range(NB):
        pltpu.make_async_copy(rows_v[b], agg_sp.at[dst_v[b]], ssem[b]).wait()

    plsc.subcore_barrier()
    pltpu.sync_copy(agg_sp.at[pl.ds(sid * ROWS_PT, ROWS_PT)],
                    out_hbm.at[cid, pl.ds(sid * ROWS_PT, ROWS_PT)])


@functools.partial(
    pl.kernel,
    out_type=jax.ShapeDtypeStruct((NC, NPAD, D), jnp.float32),
    mesh=_mesh,
    scratch_types=[
        [pltpu.VMEM((RECW,), jnp.int32)] * NB,
        [pltpu.VMEM((K,), jnp.int32)] * NB,
        [pltpu.VMEM((K,), jnp.int32)] * NB,
        [pltpu.VMEM((K, D), jnp.float32)] * NB,
        [pltpu.SemaphoreType.DMA] * NB,
        [pltpu.SemaphoreType.DMA] * NB,
        [pltpu.SemaphoreType.DMA] * NB,
        pltpu.VMEM_SHARED((NPAD, D), jnp.float32),
    ],
    compiler_params=_sc_params,
)
def _sc_msg(rec_hbm, tab_hbm, out_hbm,
            rec_v, dst_v, gidx_v, rows_v, rsem, gsem, ssem,
            agg_sp):
    _msg_body(rec_hbm, tab_hbm, out_hbm,
              rec_v, dst_v, gidx_v, rows_v, rsem, gsem, ssem,
              agg_sp)


def _fin_body(a_ref, xr_ref, b_ref, o_ref):
    o_ref[...] = jnp.tanh(a_ref[0] + a_ref[1] + xr_ref[...] + b_ref[...])


def _finish(aggp, xroot, bias):
    bm = 1000
    return pl.pallas_call(
        _fin_body,
        grid=(N // bm,),
        in_specs=[
            pl.BlockSpec((NC, bm, D), lambda i: (0, i, 0)),
            pl.BlockSpec((bm, D), lambda i: (i, 0)),
            pl.BlockSpec((1, D), lambda i: (0, 0)),
        ],
        out_specs=pl.BlockSpec((bm, D), lambda i: (i, 0)),
        out_shape=jax.ShapeDtypeStruct((N, D), jnp.float32),
    )(aggp, xroot, bias.reshape(1, D))


def _layer(h, rec, W, root, bias):
    wc = jnp.concatenate([W, root[None]], axis=0)
    xw = _mm(h, wc)
    table = xw.reshape((R + 1) * N, D)
    aggp = _sc_msg(rec, table)
    return _finish(aggp, xw[R], bias)


def kernel(x, edge_index, edge_attr, W1, root1, bias1, W2, root2, bias2):
    pk = _pack(edge_index, edge_attr)
    counts = _sc_counts(pk)
    inv = _inv(counts)
    rec = _sc_norm(pk, inv)
    h = _layer(x, rec, W1, root1, bias1)
    return _layer(h, rec, W2, root2, bias2)
